# baseline (speedup 1.0000x reference)
import functools

import jax
import jax.numpy as jnp
from jax import lax
from jax.experimental import pallas as pl
from jax.experimental.pallas import tpu as pltpu
from jax.experimental.pallas import tpu_sc as plsc

_HID = 128
_NJ = _HID // 16


def _vsum_tree(vs):
    while len(vs) > 1:
        vs = [a + b for a, b in zip(vs[::2], vs[1::2])]
    return vs[0]


def _rsqrt_scalar(v):
    i = lax.bitcast_convert_type(v, jnp.int32)
    y = lax.bitcast_convert_type(jnp.int32(0x5F3759DF) - (i >> 1), jnp.float32)
    hv = 0.5 * v
    for _ in range(3):
        y = y * (1.5 - hv * y * y)
    return y


def _lane_total(v):
    sc = [v[l] for l in range(16)]
    while len(sc) > 1:
        sc = [a + b for a, b in zip(sc[::2], sc[1::2])]
    return sc[0]


@functools.lru_cache(maxsize=None)
def _sc_fused(n_rows: int, seq: int, chunk: int):
    info = plsc.get_sparse_core_info()
    nc, ns = info.num_cores, info.num_subcores
    nw = nc * ns
    rows_per_w = n_rows // nw
    n_chunks = rows_per_w // chunk
    s_per_w = seq // nw
    chunks_per_s = n_chunks // s_per_w
    groups = chunk // 16

    mesh = plsc.VectorSubcoreMesh(core_axis_name="c", subcore_axis_name="s")

    @functools.partial(
        pl.kernel,
        mesh=mesh,
        out_type=jax.ShapeDtypeStruct((n_rows, _HID), jnp.float32),
        scratch_types=[
            pltpu.VMEM((n_chunks, chunk), jnp.int32),
            pltpu.VMEM((n_chunks, chunk), jnp.float32),
            pltpu.VMEM((s_per_w, _HID), jnp.float32),
            pltpu.VMEM((3, _HID), jnp.float32),
            pltpu.VMEM((chunk, _HID), jnp.float32),
            pltpu.VMEM((chunk, _HID), jnp.float32),
            pltpu.SemaphoreType.DMA,
            pltpu.SemaphoreType.DMA,
            pltpu.SemaphoreType.DMA,
            pltpu.SemaphoreType.DMA,
        ],
    )
    def k(table_hbm, ids_hbm, tt_hbm, c1_hbm, aux_hbm, out_hbm,
          idx_all, tt_all, c1_t, aux_v, buf0, buf1, g0, g1, s0, s1):
        wid = lax.axis_index("s") * nc + lax.axis_index("c")
        base = wid * rows_per_w
        buf = (buf0, buf1)
        gs = (g0, g1)
        ss = (s0, s1)

        pltpu.sync_copy(ids_hbm.at[wid], idx_all)
        pltpu.sync_copy(tt_hbm.at[wid], tt_all)
        pltpu.sync_copy(c1_hbm.at[pl.ds(wid * s_per_w, s_per_w)], c1_t)
        pltpu.sync_copy(aux_hbm, aux_v)
        pltpu.async_copy(table_hbm.at[idx_all.at[0]], buf0, g0)

        def compute(i, b):
            s_loc = i // chunks_per_s
            c1v = [c1_t[s_loc, pl.ds(16 * j, 16)] for j in range(_NJ)]
            dltv = [aux_v[0, pl.ds(16 * j, 16)] for j in range(_NJ)]
            gamv = [aux_v[1, pl.ds(16 * j, 16)] for j in range(_NJ)]
            betv = [aux_v[2, pl.ds(16 * j, 16)] for j in range(_NJ)]

            @plsc.parallel_loop(0, groups, unroll=2)
            def grp(g):
                r0 = g * 16
                tv = tt_all[i, pl.ds(r0, 16)]
                for l in range(16):
                    r = r0 + l
                    tsp = jnp.full((16,), tv[l])
                    x = [
                        b[r, pl.ds(16 * j, 16)] + c1v[j] + tsp * dltv[j]
                        for j in range(_NJ)
                    ]
                    mean = _lane_total(_vsum_tree(x)) * (1.0 / _HID)
                    ex2 = _lane_total(
                        _vsum_tree([v * v for v in x])) * (1.0 / _HID)
                    inv = _rsqrt_scalar(ex2 - mean * mean + 1e-5)
                    minv = jnp.full((16,), mean)
                    vinv = jnp.full((16,), inv)
                    for j in range(_NJ):
                        b[r, pl.ds(16 * j, 16)] = (
                            (x[j] - minv) * (vinv * gamv[j]) + betv[j]
                        )

        def pair(p, carry):
            for q in range(2):
                i = 2 * p + q
                cur, nxt = q, 1 - q

                @pl.when(i + 1 < n_chunks)
                def _fire():
                    @pl.when(i >= 1)
                    def _drain():
                        pltpu.make_async_copy(
                            buf[nxt], out_hbm.at[pl.ds(base, chunk)], ss[nxt]
                        ).wait()

                    pltpu.async_copy(
                        table_hbm.at[idx_all.at[i + 1]], buf[nxt], gs[nxt]
                    )

                pltpu.make_async_copy(
                    table_hbm.at[idx_all.at[i]], buf[cur], gs[cur]
                ).wait()
                compute(i, buf[cur])
                pltpu.async_copy(
                    buf[cur], out_hbm.at[pl.ds(base + i * chunk, chunk)], ss[cur]
                )
            return carry

        lax.fori_loop(0, n_chunks // 2, pair, 0)
        for bb in range(2):
            pltpu.make_async_copy(
                buf[bb], out_hbm.at[pl.ds(base, chunk)], ss[bb]
            ).wait()

    return k


def kernel(input_ids, position_ids, token_type_ids, word_emb, pos_emb,
           type_emb, ln_gamma, ln_beta):
    s, b = input_ids.shape
    h = word_emb.shape[1]
    n_rows = s * b
    chunk = 128
    info = plsc.get_sparse_core_info()
    nw = info.num_cores * info.num_subcores
    n_chunks = (n_rows // nw) // chunk

    pos_table = jnp.take(pos_emb, position_ids[0].astype(jnp.int32), axis=0)
    c1 = pos_table + type_emb[0]
    aux = jnp.stack([type_emb[1] - type_emb[0], ln_gamma, ln_beta])
    ids_t = input_ids.reshape(nw, n_chunks, chunk).astype(jnp.int32)
    tt_t = token_type_ids.reshape(nw, n_chunks, chunk).astype(jnp.float32)

    out = _sc_fused(n_rows, s, chunk)(word_emb, ids_t, tt_t, c1, aux)
    return out.reshape(s, b, h)


# speedup vs baseline: 5.0984x; 5.0984x over previous
import functools

import jax
import jax.numpy as jnp
from jax import lax
from jax.experimental import pallas as pl
from jax.experimental.pallas import tpu as pltpu
from jax.experimental.pallas import tpu_sc as plsc

_HID = 128


@functools.lru_cache(maxsize=None)
def _sc_gather(n_rows: int, chunk: int):
    info = plsc.get_sparse_core_info()
    nc, ns = info.num_cores, info.num_subcores
    nw = nc * ns
    rows_per_w = n_rows // nw
    n_chunks = rows_per_w // chunk

    mesh = plsc.VectorSubcoreMesh(core_axis_name="c", subcore_axis_name="s")

    @functools.partial(
        pl.kernel,
        mesh=mesh,
        out_type=jax.ShapeDtypeStruct((n_rows, _HID), jnp.float32),
        scratch_types=[
            pltpu.VMEM((n_chunks, chunk), jnp.int32),
            pltpu.VMEM((chunk, _HID), jnp.float32),
            pltpu.VMEM((chunk, _HID), jnp.float32),
            pltpu.SemaphoreType.DMA,
            pltpu.SemaphoreType.DMA,
            pltpu.SemaphoreType.DMA,
            pltpu.SemaphoreType.DMA,
        ],
    )
    def k(table_hbm, ids_hbm, out_hbm, idx_all, buf0, buf1, g0, g1, s0, s1):
        wid = lax.axis_index("s") * nc + lax.axis_index("c")
        base = wid * rows_per_w
        buf = (buf0, buf1)
        gs = (g0, g1)
        ss = (s0, s1)

        pltpu.sync_copy(ids_hbm.at[wid], idx_all)
        pltpu.async_copy(table_hbm.at[idx_all.at[0]], buf0, g0)

        def pair(p, carry):
            for q in range(2):
                i = 2 * p + q
                cur, nxt = q, 1 - q

                @pl.when(i + 1 < n_chunks)
                def _fire():
                    @pl.when(i >= 1)
                    def _drain():
                        pltpu.make_async_copy(
                            buf[nxt], out_hbm.at[pl.ds(base, chunk)], ss[nxt]
                        ).wait()

                    pltpu.async_copy(
                        table_hbm.at[idx_all.at[i + 1]], buf[nxt], gs[nxt]
                    )

                pltpu.make_async_copy(
                    table_hbm.at[idx_all.at[i]], buf[cur], gs[cur]
                ).wait()
                pltpu.async_copy(
                    buf[cur], out_hbm.at[pl.ds(base + i * chunk, chunk)], ss[cur]
                )
            return carry

        lax.fori_loop(0, n_chunks // 2, pair, 0)
        for b in range(2):
            pltpu.make_async_copy(
                buf[b], out_hbm.at[pl.ds(base, chunk)], ss[b]
            ).wait()

    return k


def _ln_first(x_ref, c1_ref, tt_ref, dlt_ref, g_ref, b_ref, o_ref):
    _ln_impl(x_ref, c1_ref, tt_ref, dlt_ref, g_ref, b_ref, o_ref)


def _ln_next(x_ref, c1_ref, tt_ref, dlt_ref, g_ref, b_ref, prev_ref, o_ref):
    del prev_ref
    _ln_impl(x_ref, c1_ref, tt_ref, dlt_ref, g_ref, b_ref, o_ref)


def _ln_impl(x_ref, c1_ref, tt_ref, dlt_ref, g_ref, b_ref, o_ref):
    x = (
        x_ref[...]
        + c1_ref[...][:, None, :]
        + tt_ref[...][..., None] * dlt_ref[...][None, None, :]
    )
    mean = jnp.mean(x, axis=-1, keepdims=True)
    xc = x - mean
    var = jnp.mean(xc * xc, axis=-1, keepdims=True)
    o_ref[...] = xc * lax.rsqrt(var + 1e-5) * g_ref[...] + b_ref[...]


def kernel(input_ids, position_ids, token_type_ids, word_emb, pos_emb,
           type_emb, ln_gamma, ln_beta):
    s, b = input_ids.shape
    h = word_emb.shape[1]
    chunk = 128
    sblk = 8
    nsl = 8
    s_sl = s // nsl
    rows_sl = s_sl * b
    info = plsc.get_sparse_core_info()
    nw = info.num_cores * info.num_subcores
    n_chunks = (rows_sl // nw) // chunk

    pos_table = jnp.take(pos_emb, position_ids[0].astype(jnp.int32), axis=0)
    c1 = pos_table + type_emb[0]
    dlt = type_emb[1] - type_emb[0]
    tt = token_type_ids.astype(jnp.float32)

    sc = _sc_gather(rows_sl, chunk)
    out = None
    for k in range(nsl):
        ids_k = (
            input_ids[k * s_sl:(k + 1) * s_sl]
            .reshape(nw, n_chunks, chunk)
            .astype(jnp.int32)
        )
        gath = sc(word_emb, ids_k).reshape(s_sl, b, h)

        common_specs = [
            pl.BlockSpec((sblk, b, h), lambda i: (i, 0, 0)),
            pl.BlockSpec((sblk, h), lambda i: (i, 0)),
            pl.BlockSpec((sblk, b), lambda i: (i, 0)),
            pl.BlockSpec((h,), lambda i: (0,)),
            pl.BlockSpec((h,), lambda i: (0,)),
            pl.BlockSpec((h,), lambda i: (0,)),
        ]
        common_args = (
            gath,
            lax.dynamic_slice_in_dim(c1, k * s_sl, s_sl),
            lax.dynamic_slice_in_dim(tt, k * s_sl, s_sl),
            dlt,
            ln_gamma,
            ln_beta,
        )
        out_spec = pl.BlockSpec(
            (sblk, b, h), lambda i, k=k: (i + k * (s_sl // sblk), 0, 0)
        )
        out_shape = jax.ShapeDtypeStruct((s, b, h), jnp.float32)
        if k == 0:
            out = pl.pallas_call(
                _ln_first,
                grid=(s_sl // sblk,),
                in_specs=common_specs,
                out_specs=out_spec,
                out_shape=out_shape,
            )(*common_args)
        else:
            out = pl.pallas_call(
                _ln_next,
                grid=(s_sl // sblk,),
                in_specs=common_specs + [pl.BlockSpec(memory_space=pl.ANY)],
                out_specs=out_spec,
                out_shape=out_shape,
                input_output_aliases={6: 0},
            )(*common_args, out)
    return out


# speedup vs baseline: 5.1410x; 1.0084x over previous
import functools

import jax
import jax.numpy as jnp
from jax import lax
from jax.experimental import pallas as pl
from jax.experimental.pallas import tpu as pltpu
from jax.experimental.pallas import tpu_sc as plsc

_HID = 128


@functools.lru_cache(maxsize=None)
def _sc_gather(n_rows: int, chunk: int):
    info = plsc.get_sparse_core_info()
    nc, ns = info.num_cores, info.num_subcores
    nw = nc * ns
    rows_per_w = n_rows // nw
    n_chunks = rows_per_w // chunk

    mesh = plsc.VectorSubcoreMesh(core_axis_name="c", subcore_axis_name="s")

    @functools.partial(
        pl.kernel,
        mesh=mesh,
        out_type=jax.ShapeDtypeStruct((n_rows, _HID), jnp.float32),
        scratch_types=[
            pltpu.VMEM((n_chunks, chunk), jnp.int32),
            pltpu.VMEM((chunk, _HID), jnp.float32),
            pltpu.VMEM((chunk, _HID), jnp.float32),
            pltpu.SemaphoreType.DMA,
            pltpu.SemaphoreType.DMA,
            pltpu.SemaphoreType.DMA,
            pltpu.SemaphoreType.DMA,
        ],
    )
    def k(table_hbm, ids_hbm, out_hbm, idx_all, buf0, buf1, g0, g1, s0, s1):
        wid = lax.axis_index("s") * nc + lax.axis_index("c")
        base = wid * rows_per_w
        buf = (buf0, buf1)
        gs = (g0, g1)
        ss = (s0, s1)

        pltpu.sync_copy(ids_hbm.at[wid], idx_all)
        pltpu.async_copy(table_hbm.at[idx_all.at[0]], buf0, g0)

        def pair(p, carry):
            for q in range(2):
                i = 2 * p + q
                cur, nxt = q, 1 - q

                @pl.when(i + 1 < n_chunks)
                def _fire():
                    @pl.when(i >= 1)
                    def _drain():
                        pltpu.make_async_copy(
                            buf[nxt], out_hbm.at[pl.ds(base, chunk)], ss[nxt]
                        ).wait()

                    pltpu.async_copy(
                        table_hbm.at[idx_all.at[i + 1]], buf[nxt], gs[nxt]
                    )

                pltpu.make_async_copy(
                    table_hbm.at[idx_all.at[i]], buf[cur], gs[cur]
                ).wait()
                pltpu.async_copy(
                    buf[cur], out_hbm.at[pl.ds(base + i * chunk, chunk)], ss[cur]
                )
            return carry

        lax.fori_loop(0, n_chunks // 2, pair, 0)
        for b in range(2):
            pltpu.make_async_copy(
                buf[b], out_hbm.at[pl.ds(base, chunk)], ss[b]
            ).wait()

    return k


def _ln_first(x_ref, c1_ref, tt_ref, dlt_ref, g_ref, b_ref, o_ref):
    _ln_impl(x_ref, c1_ref, tt_ref, dlt_ref, g_ref, b_ref, o_ref)


def _ln_next(x_ref, c1_ref, tt_ref, dlt_ref, g_ref, b_ref, prev_ref, o_ref):
    del prev_ref
    _ln_impl(x_ref, c1_ref, tt_ref, dlt_ref, g_ref, b_ref, o_ref)


def _ln_impl(x_ref, c1_ref, tt_ref, dlt_ref, g_ref, b_ref, o_ref):
    x = (
        x_ref[...]
        + c1_ref[...][:, None, :]
        + tt_ref[...][..., None] * dlt_ref[...][None, None, :]
    )
    mean = jnp.mean(x, axis=-1, keepdims=True)
    xc = x - mean
    var = jnp.mean(xc * xc, axis=-1, keepdims=True)
    o_ref[...] = xc * lax.rsqrt(var + 1e-5) * g_ref[...] + b_ref[...]


def kernel(input_ids, position_ids, token_type_ids, word_emb, pos_emb,
           type_emb, ln_gamma, ln_beta):
    s, b = input_ids.shape
    h = word_emb.shape[1]
    chunk = 128
    sblk = 8
    nsl = 4
    s_sl = s // nsl
    rows_sl = s_sl * b
    info = plsc.get_sparse_core_info()
    nw = info.num_cores * info.num_subcores
    n_chunks = (rows_sl // nw) // chunk

    pos_table = jnp.take(pos_emb, position_ids[0].astype(jnp.int32), axis=0)
    c1 = pos_table + type_emb[0]
    dlt = type_emb[1] - type_emb[0]
    tt = token_type_ids.astype(jnp.float32)

    sc = _sc_gather(rows_sl, chunk)
    out = None
    for k in range(nsl):
        ids_k = (
            input_ids[k * s_sl:(k + 1) * s_sl]
            .reshape(nw, n_chunks, chunk)
            .astype(jnp.int32)
        )
        gath = sc(word_emb, ids_k).reshape(s_sl, b, h)

        common_specs = [
            pl.BlockSpec((sblk, b, h), lambda i: (i, 0, 0)),
            pl.BlockSpec((sblk, h), lambda i: (i, 0)),
            pl.BlockSpec((sblk, b), lambda i: (i, 0)),
            pl.BlockSpec((h,), lambda i: (0,)),
            pl.BlockSpec((h,), lambda i: (0,)),
            pl.BlockSpec((h,), lambda i: (0,)),
        ]
        common_args = (
            gath,
            lax.dynamic_slice_in_dim(c1, k * s_sl, s_sl),
            lax.dynamic_slice_in_dim(tt, k * s_sl, s_sl),
            dlt,
            ln_gamma,
            ln_beta,
        )
        out_spec = pl.BlockSpec(
            (sblk, b, h), lambda i, k=k: (i + k * (s_sl // sblk), 0, 0)
        )
        out_shape = jax.ShapeDtypeStruct((s, b, h), jnp.float32)
        if k == 0:
            out = pl.pallas_call(
                _ln_first,
                grid=(s_sl // sblk,),
                in_specs=common_specs,
                out_specs=out_spec,
                out_shape=out_shape,
            )(*common_args)
        else:
            out = pl.pallas_call(
                _ln_next,
                grid=(s_sl // sblk,),
                in_specs=common_specs + [pl.BlockSpec(memory_space=pl.ANY)],
                out_specs=out_spec,
                out_shape=out_shape,
                input_output_aliases={6: 0},
            )(*common_args, out)
    return out
